# baseline (speedup 1.0000x reference)
import functools

import jax
import jax.numpy as jnp
from jax import lax
from jax.experimental import pallas as pl
from jax.experimental.pallas import tpu as pltpu
from jax.experimental.pallas import tpu_sc as plsc

_NC = 2
_NS = 16
_NW = _NC * _NS


_BR = 1024


def _flatten_body(xt_ref, o_ref):
    xt = xt_ref[...]
    parts = [jnp.transpose(xt[:, j * _BR:(j + 1) * _BR]) for j in range(8)]
    o_ref[...] = jnp.concatenate(parts, axis=1)


def _flatten_table(table_t):
    D, V = table_t.shape
    grid = (V * D // 128 + _BR - 1) // _BR
    rows = grid * _BR
    out = pl.pallas_call(
        _flatten_body,
        grid=(grid,),
        in_specs=[pl.BlockSpec((D, 8 * _BR), lambda i: (0, i))],
        out_specs=pl.BlockSpec((_BR, 128), lambda i: (i, 0)),
        out_shape=jax.ShapeDtypeStruct((rows, 128), jnp.float32),
    )(table_t)
    return jnp.reshape(out, (rows * 128,))


def _gather_rows(flat, idx, D):
    B = idx.shape[0]
    bpw = B // _NW
    CH = 16
    NCH = bpw // CH
    mesh = plsc.VectorSubcoreMesh(core_axis_name="c", subcore_axis_name="s")

    @functools.partial(
        pl.kernel,
        mesh=mesh,
        out_type=jax.ShapeDtypeStruct((B, D), jnp.float32),
        scratch_types=[
            pltpu.VMEM((bpw,), jnp.int32),
            pltpu.VMEM((bpw, D), jnp.float32),
            pltpu.SemaphoreType.DMA,
        ],
    )
    def k(flat_hbm, idx_hbm, out_hbm, idx_v, rows_v, sem):
        wid = lax.axis_index("s") * _NC + lax.axis_index("c")
        base = wid * bpw
        pltpu.sync_copy(idx_hbm.at[pl.ds(base, bpw)], idx_v)

        def chunk(c):
            v = idx_v[pl.ds(c * CH, CH)]
            off = (lax.shift_left(lax.shift_right_logical(v, 13), 17)
                   + lax.shift_left(lax.bitwise_and(v, 1023), 7)
                   + lax.shift_left(
                       lax.bitwise_and(lax.shift_right_logical(v, 10), 7), 4))
            copies = []
            for k_ in range(CH):
                copies.append(pltpu.async_copy(
                    flat_hbm.at[pl.ds(pl.multiple_of(off[k_], 8), D)],
                    rows_v.at[c * CH + k_], sem))
            for cp in copies:
                cp.wait()

        for c_ in range(NCH):
            chunk(c_)
        pltpu.sync_copy(rows_v, out_hbm.at[pl.ds(base, bpw)])

    return k(flat, idx)


def _mlp_body(x_ref, w1_ref, b1_ref, g1_ref, be1_ref,
              w2_ref, b2_ref, g2_ref, be2_ref, w3_ref, b3_ref, o_ref):
    hi = jax.lax.Precision.DEFAULT
    x = x_ref[...]
    h = lax.dot_general(x, w1_ref[...], (((1,), (0,)), ((), ())),
                        preferred_element_type=jnp.float32, precision=hi)
    h = h + b1_ref[...][None, :]
    mean = jnp.mean(h, axis=0)
    var = jnp.mean(h * h, axis=0) - mean * mean
    s = g1_ref[...] * lax.rsqrt(var + 1e-5)
    h = jnp.maximum(h * s[None, :] + (be1_ref[...] - mean * s)[None, :], 0.0)

    h2 = lax.dot_general(h, w2_ref[...], (((1,), (0,)), ((), ())),
                         preferred_element_type=jnp.float32, precision=hi)
    h2 = h2 + b2_ref[...][None, :]
    mean2 = jnp.mean(h2, axis=0)
    var2 = jnp.mean(h2 * h2, axis=0) - mean2 * mean2
    s2 = g2_ref[...] * lax.rsqrt(var2 + 1e-5)
    h2 = jnp.maximum(h2 * s2[None, :] + (be2_ref[...] - mean2 * s2)[None, :], 0.0)

    o = jnp.sum(h2 * w3_ref[...], axis=1, keepdims=True)
    o_ref[...] = o + b3_ref[...]


def kernel(x_numeric, x_diag_cat, table, W1, b1, g1, be1, W2, b2, g2, be2, W3, b3):
    B, F = x_numeric.shape
    D = table.shape[1]
    flat = _flatten_table(jnp.transpose(table))
    emb = _gather_rows(flat, jnp.reshape(x_diag_cat, (B,)), D)

    K = F + D
    Kp = 32
    x = jnp.concatenate(
        [x_numeric, emb, jnp.zeros((B, Kp - K), jnp.float32)], axis=1)
    w1p = jnp.zeros((Kp, W1.shape[0]), jnp.float32).at[:K, :].set(W1.T)

    out = pl.pallas_call(
        _mlp_body,
        out_shape=jax.ShapeDtypeStruct((B, 1), jnp.float32),
    )(x, w1p, b1, g1, be1, W2.T, b2, g2, be2, jnp.reshape(W3, (1, 128)),
      jnp.reshape(b3, (1, 1)))
    return out


# speedup vs baseline: 1.9181x; 1.9181x over previous
import functools

import jax
import jax.numpy as jnp
from jax import lax
from jax.experimental import pallas as pl
from jax.experimental.pallas import tpu as pltpu
from jax.experimental.pallas import tpu_sc as plsc

_NC = 2
_NS = 16
_NW = _NC * _NS


_BR = 1024


def _flatten_body(xt_ref, o_ref):
    xt = xt_ref[...]
    parts = [xt[:, j * _BR:(j + 1) * _BR] for j in range(8)]
    xbig = jnp.concatenate(parts, axis=0)
    eye = jnp.eye(128, dtype=jnp.float32)
    o_ref[...] = lax.dot_general(xbig, eye, (((0,), (0,)), ((), ())),
                                 preferred_element_type=jnp.float32)


def _flatten_table(table_t):
    D, V = table_t.shape
    grid = (V * D // 128 + _BR - 1) // _BR
    rows = grid * _BR
    out = pl.pallas_call(
        _flatten_body,
        grid=(grid,),
        in_specs=[pl.BlockSpec((D, 8 * _BR), lambda i: (0, i))],
        out_specs=pl.BlockSpec((_BR, 128), lambda i: (i, 0)),
        out_shape=jax.ShapeDtypeStruct((rows, 128), jnp.float32),
    )(table_t)
    return jnp.reshape(out, (rows * 128,))


def _gather_rows(flat, idx, D):
    B = idx.shape[0]
    bpw = B // _NW
    CH = 16
    NCH = bpw // CH
    mesh = plsc.VectorSubcoreMesh(core_axis_name="c", subcore_axis_name="s")

    @functools.partial(
        pl.kernel,
        mesh=mesh,
        out_type=jax.ShapeDtypeStruct((B, D), jnp.float32),
        scratch_types=[
            pltpu.VMEM((bpw,), jnp.int32),
            pltpu.VMEM((bpw, D), jnp.float32),
            pltpu.SemaphoreType.DMA,
        ],
    )
    def k(flat_hbm, idx_hbm, out_hbm, idx_v, rows_v, sem):
        wid = lax.axis_index("s") * _NC + lax.axis_index("c")
        base = wid * bpw
        pltpu.sync_copy(idx_hbm.at[pl.ds(base, bpw)], idx_v)

        def chunk(c):
            v = idx_v[pl.ds(c * CH, CH)]
            off = (lax.shift_left(lax.shift_right_logical(v, 13), 17)
                   + lax.shift_left(lax.bitwise_and(v, 1023), 7)
                   + lax.shift_left(
                       lax.bitwise_and(lax.shift_right_logical(v, 10), 7), 4))
            copies = []
            for k_ in range(CH):
                copies.append(pltpu.async_copy(
                    flat_hbm.at[pl.ds(pl.multiple_of(off[k_], 8), D)],
                    rows_v.at[c * CH + k_], sem))
            for cp in copies:
                cp.wait()

        for c_ in range(NCH):
            chunk(c_)
        pltpu.sync_copy(rows_v, out_hbm.at[pl.ds(base, bpw)])

    return k(flat, idx)


def _mlp_body(x_ref, w1_ref, b1_ref, g1_ref, be1_ref,
              w2_ref, b2_ref, g2_ref, be2_ref, w3_ref, b3_ref, o_ref):
    hi = jax.lax.Precision.DEFAULT
    x = x_ref[...]
    h = lax.dot_general(x, w1_ref[...], (((1,), (0,)), ((), ())),
                        preferred_element_type=jnp.float32, precision=hi)
    h = h + b1_ref[...][None, :]
    mean = jnp.mean(h, axis=0)
    var = jnp.mean(h * h, axis=0) - mean * mean
    s = g1_ref[...] * lax.rsqrt(var + 1e-5)
    h = jnp.maximum(h * s[None, :] + (be1_ref[...] - mean * s)[None, :], 0.0)

    h2 = lax.dot_general(h, w2_ref[...], (((1,), (0,)), ((), ())),
                         preferred_element_type=jnp.float32, precision=hi)
    h2 = h2 + b2_ref[...][None, :]
    mean2 = jnp.mean(h2, axis=0)
    var2 = jnp.mean(h2 * h2, axis=0) - mean2 * mean2
    s2 = g2_ref[...] * lax.rsqrt(var2 + 1e-5)
    h2 = jnp.maximum(h2 * s2[None, :] + (be2_ref[...] - mean2 * s2)[None, :], 0.0)

    o = jnp.sum(h2 * w3_ref[...], axis=1, keepdims=True)
    o_ref[...] = o + b3_ref[...]


def kernel(x_numeric, x_diag_cat, table, W1, b1, g1, be1, W2, b2, g2, be2, W3, b3):
    B, F = x_numeric.shape
    D = table.shape[1]
    flat = _flatten_table(jnp.transpose(table))
    emb = _gather_rows(flat, jnp.reshape(x_diag_cat, (B,)), D)

    K = F + D
    Kp = 32
    x = jnp.concatenate(
        [x_numeric, emb, jnp.zeros((B, Kp - K), jnp.float32)], axis=1)
    w1p = jnp.zeros((Kp, W1.shape[0]), jnp.float32).at[:K, :].set(W1.T)

    out = pl.pallas_call(
        _mlp_body,
        out_shape=jax.ShapeDtypeStruct((B, 1), jnp.float32),
    )(x, w1p, b1, g1, be1, W2.T, b2, g2, be2, jnp.reshape(W3, (1, 128)),
      jnp.reshape(b3, (1, 1)))
    return out


# speedup vs baseline: 2.0889x; 1.0890x over previous
import functools

import jax
import jax.numpy as jnp
from jax import lax
from jax.experimental import pallas as pl
from jax.experimental.pallas import tpu as pltpu
from jax.experimental.pallas import tpu_sc as plsc

_NC = 2
_NS = 16
_NW = _NC * _NS


_BR = 1024


def _flatten_body(xt_ref, o_ref):
    xt = xt_ref[...]
    parts = [xt[:, j * _BR:(j + 1) * _BR] for j in range(8)]
    xbig = jnp.concatenate(parts, axis=0)
    eye = jnp.eye(128, dtype=jnp.float32)
    o_ref[...] = lax.dot_general(xbig, eye, (((0,), (0,)), ((), ())),
                                 preferred_element_type=jnp.float32)


def _flatten_table(table_t):
    D, V = table_t.shape
    grid = (V * D // 128 + _BR - 1) // _BR
    rows = grid * _BR
    out = pl.pallas_call(
        _flatten_body,
        grid=(grid,),
        in_specs=[pl.BlockSpec((D, 8 * _BR), lambda i: (0, i))],
        out_specs=pl.BlockSpec((_BR, 128), lambda i: (i, 0)),
        out_shape=jax.ShapeDtypeStruct((rows, 128), jnp.float32),
    )(table_t)
    return jnp.reshape(out, (rows * 128,))


def _gather_rows(flat, idx, D):
    B = idx.shape[0]
    bpw = B // _NW
    CH = 16
    NCH = bpw // CH
    mesh = plsc.VectorSubcoreMesh(core_axis_name="c", subcore_axis_name="s")

    @functools.partial(
        pl.kernel,
        mesh=mesh,
        out_type=jax.ShapeDtypeStruct((B, D), jnp.float32),
        scratch_types=[
            pltpu.VMEM((bpw,), jnp.int32),
            pltpu.VMEM((bpw, D), jnp.float32),
            pltpu.SemaphoreType.DMA,
        ],
    )
    def k(flat_hbm, idx_hbm, out_hbm, idx_v, rows_v, sem):
        wid = lax.axis_index("s") * _NC + lax.axis_index("c")
        base = wid * bpw
        pltpu.sync_copy(idx_hbm.at[pl.ds(base, bpw)], idx_v)

        def fire(c):
            v = idx_v[pl.ds(c * CH, CH)]
            off = (lax.shift_left(lax.shift_right_logical(v, 13), 17)
                   + lax.shift_left(lax.bitwise_and(v, 1023), 7)
                   + lax.shift_left(
                       lax.bitwise_and(lax.shift_right_logical(v, 10), 7), 4))
            return [pltpu.async_copy(
                flat_hbm.at[pl.ds(pl.multiple_of(off[k_], 8), D)],
                rows_v.at[c * CH + k_], sem) for k_ in range(CH)]

        pend = fire(0)
        for c_ in range(1, NCH):
            nxt = fire(c_)
            for cp in pend:
                cp.wait()
            pend = nxt
        for cp in pend:
            cp.wait()
        pltpu.sync_copy(rows_v, out_hbm.at[pl.ds(base, bpw)])

    return k(flat, idx)


def _mlp_body(xnt_ref, emb_ref, w1n_ref, w1e_ref, b1_ref, g1_ref, be1_ref,
              w2_ref, b2_ref, g2_ref, be2_ref, w3_ref, b3_ref, o_ref):
    hi = jax.lax.Precision.DEFAULT
    h = (lax.dot_general(xnt_ref[...], w1n_ref[...], (((0,), (0,)), ((), ())),
                         preferred_element_type=jnp.float32, precision=hi)
         + lax.dot_general(emb_ref[...], w1e_ref[...], (((1,), (0,)), ((), ())),
                           preferred_element_type=jnp.float32, precision=hi))
    h = h + b1_ref[...][None, :]
    mean = jnp.mean(h, axis=0)
    var = jnp.mean(h * h, axis=0) - mean * mean
    s = g1_ref[...] * lax.rsqrt(var + 1e-5)
    h = jnp.maximum(h * s[None, :] + (be1_ref[...] - mean * s)[None, :], 0.0)

    h2 = lax.dot_general(h, w2_ref[...], (((1,), (0,)), ((), ())),
                         preferred_element_type=jnp.float32, precision=hi)
    h2 = h2 + b2_ref[...][None, :]
    mean2 = jnp.mean(h2, axis=0)
    var2 = jnp.mean(h2 * h2, axis=0) - mean2 * mean2
    s2 = g2_ref[...] * lax.rsqrt(var2 + 1e-5)
    h2 = jnp.maximum(h2 * s2[None, :] + (be2_ref[...] - mean2 * s2)[None, :], 0.0)

    o = jnp.sum(h2 * w3_ref[...], axis=1, keepdims=True)
    o_ref[...] = o + b3_ref[...]


def kernel(x_numeric, x_diag_cat, table, W1, b1, g1, be1, W2, b2, g2, be2, W3, b3):
    B, F = x_numeric.shape
    D = table.shape[1]
    flat = _flatten_table(jnp.transpose(table))
    emb = _gather_rows(flat, jnp.reshape(x_diag_cat, (B,)), D)

    out = pl.pallas_call(
        _mlp_body,
        out_shape=jax.ShapeDtypeStruct((B, 1), jnp.float32),
    )(jnp.transpose(x_numeric), emb, W1[:, :F].T, W1[:, F:].T,
      b1, g1, be1, W2.T, b2, g2, be2, jnp.reshape(W3, (1, 128)),
      jnp.reshape(b3, (1, 1)))
    return out


# speedup vs baseline: 3.0470x; 1.4587x over previous
import functools

import jax
import jax.numpy as jnp
from jax import lax
from jax.experimental import pallas as pl
from jax.experimental.pallas import tpu as pltpu
from jax.experimental.pallas import tpu_sc as plsc

_NC = 2
_NS = 16
_NW = _NC * _NS


_BR = 4096
_LB = _BR.bit_length() - 1


def _flatten_body(xt_ref, o_ref):
    xt = xt_ref[...]
    parts = [xt[:, j * _BR:(j + 1) * _BR] for j in range(8)]
    xbig = jnp.concatenate(parts, axis=0)
    eye = jnp.eye(128, dtype=jnp.float32)
    o_ref[...] = lax.dot_general(xbig, eye, (((0,), (0,)), ((), ())),
                                 preferred_element_type=jnp.float32)


def _flatten_table(table_t):
    D, V = table_t.shape
    grid = (V * D // 128 + _BR - 1) // _BR
    rows = grid * _BR
    out = pl.pallas_call(
        _flatten_body,
        grid=(grid,),
        in_specs=[pl.BlockSpec((D, 8 * _BR), lambda i: (0, i))],
        out_specs=pl.BlockSpec((_BR, 128), lambda i: (i, 0)),
        out_shape=jax.ShapeDtypeStruct((rows, 128), jnp.float32),
    )(table_t)
    return jnp.reshape(out, (rows * 128,))


def _gather_rows(flat, idx, D):
    B = idx.shape[0]
    bpw = B // _NW
    CH = 16
    NCH = bpw // CH
    mesh = plsc.VectorSubcoreMesh(core_axis_name="c", subcore_axis_name="s")

    @functools.partial(
        pl.kernel,
        mesh=mesh,
        out_type=jax.ShapeDtypeStruct((B, D), jnp.float32),
        scratch_types=[
            pltpu.VMEM((bpw,), jnp.int32),
            pltpu.VMEM((bpw, D), jnp.float32),
            pltpu.SemaphoreType.DMA,
        ],
    )
    def k(flat_hbm, idx_hbm, out_hbm, idx_v, rows_v, sem):
        wid = lax.axis_index("s") * _NC + lax.axis_index("c")
        base = wid * bpw
        pltpu.sync_copy(idx_hbm.at[pl.ds(base, bpw)], idx_v)

        def fire(c):
            v = idx_v[pl.ds(c * CH, CH)]
            off = (lax.shift_left(lax.shift_right_logical(v, _LB + 3), _LB + 7)
                   + lax.shift_left(lax.bitwise_and(v, _BR - 1), 7)
                   + lax.shift_left(
                       lax.bitwise_and(lax.shift_right_logical(v, _LB), 7), 4))
            return [pltpu.async_copy(
                flat_hbm.at[pl.ds(pl.multiple_of(off[k_], 8), D)],
                rows_v.at[c * CH + k_], sem) for k_ in range(CH)]

        pend = fire(0)
        for c_ in range(1, NCH):
            nxt = fire(c_)
            for cp in pend:
                cp.wait()
            pend = nxt
        for cp in pend:
            cp.wait()
        pltpu.sync_copy(rows_v, out_hbm.at[pl.ds(base, bpw)])

    return k(flat, idx)


def _mlp_body(xnt_ref, emb_ref, w1n_ref, w1e_ref, b1_ref, g1_ref, be1_ref,
              w2_ref, b2_ref, g2_ref, be2_ref, w3_ref, b3_ref, o_ref):
    hi = jax.lax.Precision.DEFAULT
    h = (lax.dot_general(xnt_ref[...], w1n_ref[...], (((0,), (0,)), ((), ())),
                         preferred_element_type=jnp.float32, precision=hi)
         + lax.dot_general(emb_ref[...], w1e_ref[...], (((1,), (0,)), ((), ())),
                           preferred_element_type=jnp.float32, precision=hi))
    h = h + b1_ref[...][None, :]
    mean = jnp.mean(h, axis=0)
    var = jnp.mean(h * h, axis=0) - mean * mean
    s = g1_ref[...] * lax.rsqrt(var + 1e-5)
    h = jnp.maximum(h * s[None, :] + (be1_ref[...] - mean * s)[None, :], 0.0)

    h2 = lax.dot_general(h, w2_ref[...], (((1,), (0,)), ((), ())),
                         preferred_element_type=jnp.float32, precision=hi)
    h2 = h2 + b2_ref[...][None, :]
    mean2 = jnp.mean(h2, axis=0)
    var2 = jnp.mean(h2 * h2, axis=0) - mean2 * mean2
    s2 = g2_ref[...] * lax.rsqrt(var2 + 1e-5)
    h2 = jnp.maximum(h2 * s2[None, :] + (be2_ref[...] - mean2 * s2)[None, :], 0.0)

    o = jnp.sum(h2 * w3_ref[...], axis=1, keepdims=True)
    o_ref[...] = o + b3_ref[...]


def kernel(x_numeric, x_diag_cat, table, W1, b1, g1, be1, W2, b2, g2, be2, W3, b3):
    B, F = x_numeric.shape
    D = table.shape[1]
    flat = _flatten_table(jnp.transpose(table))
    emb = _gather_rows(flat, jnp.reshape(x_diag_cat, (B,)), D)

    out = pl.pallas_call(
        _mlp_body,
        out_shape=jax.ShapeDtypeStruct((B, 1), jnp.float32),
    )(jnp.transpose(x_numeric), emb, W1[:, :F].T, W1[:, F:].T,
      b1, g1, be1, W2.T, b2, g2, be2, jnp.reshape(W3, (1, 128)),
      jnp.reshape(b3, (1, 1)))
    return out


# speedup vs baseline: 3.2416x; 1.0638x over previous
import functools

import jax
import jax.numpy as jnp
from jax import lax
from jax.experimental import pallas as pl
from jax.experimental.pallas import tpu as pltpu
from jax.experimental.pallas import tpu_sc as plsc

_NC = 2
_NS = 16
_NW = _NC * _NS


_BR = 8192
_LB = _BR.bit_length() - 1


def _flatten_body(xt_ref, o_ref):
    xt = xt_ref[...]
    parts = [xt[:, j * _BR:(j + 1) * _BR] for j in range(8)]
    xbig = jnp.concatenate(parts, axis=0)
    eye = jnp.eye(128, dtype=jnp.float32)
    o_ref[...] = lax.dot_general(xbig, eye, (((0,), (0,)), ((), ())),
                                 preferred_element_type=jnp.float32)


def _flatten_table(table_t):
    D, V = table_t.shape
    grid = (V * D // 128 + _BR - 1) // _BR
    rows = grid * _BR
    out = pl.pallas_call(
        _flatten_body,
        grid=(grid,),
        in_specs=[pl.BlockSpec((D, 8 * _BR), lambda i: (0, i))],
        out_specs=pl.BlockSpec((_BR, 128), lambda i: (i, 0)),
        out_shape=jax.ShapeDtypeStruct((rows, 128), jnp.float32),
    )(table_t)
    return jnp.reshape(out, (rows * 128,))


def _gather_rows(flat, idx, D):
    B = idx.shape[0]
    bpw = B // _NW
    CH = 16
    NCH = bpw // CH
    mesh = plsc.VectorSubcoreMesh(core_axis_name="c", subcore_axis_name="s")

    @functools.partial(
        pl.kernel,
        mesh=mesh,
        out_type=jax.ShapeDtypeStruct((B, D), jnp.float32),
        scratch_types=[
            pltpu.VMEM((bpw,), jnp.int32),
            pltpu.VMEM((bpw, D), jnp.float32),
            pltpu.SemaphoreType.DMA,
        ],
    )
    def k(flat_hbm, idx_hbm, out_hbm, idx_v, rows_v, sem):
        wid = lax.axis_index("s") * _NC + lax.axis_index("c")
        base = wid * bpw
        pltpu.sync_copy(idx_hbm.at[pl.ds(base, bpw)], idx_v)

        def fire(c):
            v = idx_v[pl.ds(c * CH, CH)]
            off = (lax.shift_left(lax.shift_right_logical(v, _LB + 3), _LB + 7)
                   + lax.shift_left(lax.bitwise_and(v, _BR - 1), 7)
                   + lax.shift_left(
                       lax.bitwise_and(lax.shift_right_logical(v, _LB), 7), 4))
            return [pltpu.async_copy(
                flat_hbm.at[pl.ds(pl.multiple_of(off[k_], 8), D)],
                rows_v.at[c * CH + k_], sem) for k_ in range(CH)]

        pend = fire(0)
        for c_ in range(1, NCH):
            nxt = fire(c_)
            for cp in pend:
                cp.wait()
            pend = nxt
        for cp in pend:
            cp.wait()
        pltpu.sync_copy(rows_v, out_hbm.at[pl.ds(base, bpw)])

    return k(flat, idx)


def _mlp_body(xnt_ref, emb_ref, w1n_ref, w1e_ref, b1_ref, g1_ref, be1_ref,
              w2_ref, b2_ref, g2_ref, be2_ref, w3_ref, b3_ref, o_ref):
    hi = jax.lax.Precision.DEFAULT
    h = (lax.dot_general(xnt_ref[...], w1n_ref[...], (((0,), (0,)), ((), ())),
                         preferred_element_type=jnp.float32, precision=hi)
         + lax.dot_general(emb_ref[...], w1e_ref[...], (((1,), (0,)), ((), ())),
                           preferred_element_type=jnp.float32, precision=hi))
    h = h + b1_ref[...][None, :]
    mean = jnp.mean(h, axis=0)
    var = jnp.mean(h * h, axis=0) - mean * mean
    s = g1_ref[...] * lax.rsqrt(var + 1e-5)
    h = jnp.maximum(h * s[None, :] + (be1_ref[...] - mean * s)[None, :], 0.0)

    h2 = lax.dot_general(h, w2_ref[...], (((1,), (0,)), ((), ())),
                         preferred_element_type=jnp.float32, precision=hi)
    h2 = h2 + b2_ref[...][None, :]
    mean2 = jnp.mean(h2, axis=0)
    var2 = jnp.mean(h2 * h2, axis=0) - mean2 * mean2
    s2 = g2_ref[...] * lax.rsqrt(var2 + 1e-5)
    h2 = jnp.maximum(h2 * s2[None, :] + (be2_ref[...] - mean2 * s2)[None, :], 0.0)

    o = jnp.sum(h2 * w3_ref[...], axis=1, keepdims=True)
    o_ref[...] = o + b3_ref[...]


def kernel(x_numeric, x_diag_cat, table, W1, b1, g1, be1, W2, b2, g2, be2, W3, b3):
    B, F = x_numeric.shape
    D = table.shape[1]
    flat = _flatten_table(jnp.transpose(table))
    emb = _gather_rows(flat, jnp.reshape(x_diag_cat, (B,)), D)

    out = pl.pallas_call(
        _mlp_body,
        out_shape=jax.ShapeDtypeStruct((B, 1), jnp.float32),
    )(jnp.transpose(x_numeric), emb, W1[:, :F].T, W1[:, F:].T,
      b1, g1, be1, W2.T, b2, g2, be2, jnp.reshape(W3, (1, 128)),
      jnp.reshape(b3, (1, 1)))
    return out
